# baseline (speedup 1.0000x reference)
import functools

import jax
import jax.numpy as jnp
from jax import lax
from jax.experimental import pallas as pl
from jax.experimental.pallas import tpu as pltpu
from jax.experimental.pallas import tpu_sc as plsc

D_MODEL = 1024
BATCH = 4
SEQ_LEN = 4096
NUM_ROWS = 16384


@jax.jit
def _sc_embed(idx, table):
    info = plsc.get_sparse_core_info()
    nc, ns = info.num_cores, info.num_subcores
    nw = nc * ns
    per_w = NUM_ROWS // nw
    assert idx.shape == (BATCH, SEQ_LEN)
    assert table.shape == (3, D_MODEL)

    mesh = plsc.VectorSubcoreMesh(core_axis_name="c", subcore_axis_name="s")

    @functools.partial(
        pl.kernel,
        mesh=mesh,
        out_type=jax.ShapeDtypeStruct((BATCH, SEQ_LEN, D_MODEL), jnp.float32),
        scratch_types=(
            pltpu.VMEM((per_w,), jnp.int32),
            pltpu.VMEM((3, D_MODEL), jnp.float32),
            pltpu.SemaphoreType.DMA,
        ),
    )
    def k(idx_hbm, tbl_hbm, out_hbm, idx_v, tbl_v, sem):
        wid = lax.axis_index("s") * nc + lax.axis_index("c")
        wpb = SEQ_LEN // per_w
        bi = wid // wpb
        sbase = (wid % wpb) * per_w
        pltpu.sync_copy(idx_hbm.at[bi, pl.ds(sbase, per_w)], idx_v)
        pltpu.sync_copy(tbl_hbm, tbl_v)

        def issue(i, _):
            s = idx_v[pl.ds(i, 1)][0]
            pltpu.async_copy(
                tbl_v.at[s],
                out_hbm.at[bi, sbase + i],
                sem)
            return 0

        lax.fori_loop(0, per_w, issue, 0)

        def drain(i, _):
            pltpu.make_async_copy(
                tbl_v.at[0],
                out_hbm.at[bi, sbase],
                sem).wait()
            return 0

        lax.fori_loop(0, per_w, drain, 0)

    return k(idx, table)


def kernel(segment_input, table):
    return _sc_embed(segment_input, table)


# speedup vs baseline: 1.3048x; 1.3048x over previous
import functools

import jax
import jax.numpy as jnp
from jax import lax
from jax.experimental import pallas as pl
from jax.experimental.pallas import tpu as pltpu
from jax.experimental.pallas import tpu_sc as plsc

D_MODEL = 1024
BATCH = 4
SEQ_LEN = 4096
NUM_ROWS = 16384


@jax.jit
def _sc_embed(idx2, tbl_r):
    info = plsc.get_sparse_core_info()
    nc, ns = info.num_cores, info.num_subcores
    nw = nc * ns
    per_w = NUM_ROWS // nw
    assert idx2.shape == (nw, per_w)
    assert tbl_r.shape == (nw, 3 * D_MODEL)

    mesh = plsc.VectorSubcoreMesh(core_axis_name="c", subcore_axis_name="s")

    @functools.partial(
        pl.kernel,
        mesh=mesh,
        out_type=jax.ShapeDtypeStruct((BATCH, SEQ_LEN, D_MODEL), jnp.float32),
        scratch_types=(
            pltpu.VMEM((per_w,), jnp.int32),
            pltpu.VMEM((3 * D_MODEL,), jnp.float32),
            pltpu.SemaphoreType.DMA,
        ),
    )
    def k(idx_hbm, tbl_hbm, out_hbm, idx_v, tbl_v, sem):
        wid = lax.axis_index("s") * nc + lax.axis_index("c")
        wpb = SEQ_LEN // per_w
        bi = wid // wpb
        sbase = (wid % wpb) * per_w
        pltpu.sync_copy(idx_hbm.at[wid], idx_v)
        pltpu.sync_copy(tbl_hbm.at[wid], tbl_v)

        def issue(i, _):
            s = idx_v[pl.ds(i, 1)][0]
            pltpu.async_copy(
                tbl_v.at[pl.ds(s * D_MODEL, D_MODEL)],
                out_hbm.at[bi, sbase + i],
                sem)
            return 0

        lax.fori_loop(0, per_w, issue, 0)

        pltpu.make_async_copy(
            out_hbm.at[bi, pl.ds(sbase, per_w)],
            out_hbm.at[bi, pl.ds(sbase, per_w)],
            sem).wait()

    return k(idx2, tbl_r)


def kernel(segment_input, table):
    info = plsc.get_sparse_core_info()
    nw = info.num_cores * info.num_subcores
    per_w = NUM_ROWS // nw
    idx2 = segment_input.astype(jnp.int32).reshape(nw, per_w)
    tbl_r = jnp.tile(table.reshape(1, -1), (nw, 1))
    return _sc_embed(idx2, tbl_r)
